# baseline (speedup 1.0000x reference)
import functools

import jax
import jax.numpy as jnp
from jax import lax
from jax.experimental import pallas as pl
from jax.experimental.pallas import tpu as pltpu
from jax.experimental.pallas import tpu_sc as plsc

N = 10000
E = 320000
D = 128
H = 128
C = 64

NC = 2
NS = 16
NW = NC * NS
EW = E // NW
CHUNK = 128
NCHUNK = 80
EWP = NCHUNK * CHUNK
WIN = 8
NWIN = NCHUNK // WIN
NP = 10240
RPT = NP // NS

_MESH = plsc.VectorSubcoreMesh(
    core_axis_name="c", subcore_axis_name="s", num_cores=NC, num_subcores=NS)


@functools.partial(
    pl.kernel,
    out_type=jax.ShapeDtypeStruct((NC * NP,), jnp.float32),
    mesh=_MESH,
    scratch_types=[
        pltpu.VMEM((NCHUNK, CHUNK), jnp.int32),
        pltpu.VMEM((CHUNK,), jnp.float32),
        pltpu.VMEM_SHARED((NP,), jnp.float32),
    ],
)
def _deg_sc(dst_hbm, ones_hbm, zeros_hbm, out_hbm, idx_d, ones_v, acc):
    c = lax.axis_index("c")
    s = lax.axis_index("s")
    w = c * NS + s
    pltpu.sync_copy(ones_hbm, ones_v)
    pltpu.sync_copy(dst_hbm.at[pl.ds(w * NCHUNK, NCHUNK)], idx_d)
    pltpu.sync_copy(zeros_hbm.at[pl.ds(s * RPT, RPT)], acc.at[pl.ds(s * RPT, RPT)])
    plsc.subcore_barrier()

    def body(i, carry):
        pltpu.sync_copy(ones_v, acc.at[idx_d.at[i]], add=True)
        return carry

    lax.fori_loop(0, NCHUNK, body, 0)
    plsc.subcore_barrier()
    pltpu.sync_copy(acc.at[pl.ds(s * RPT, RPT)],
                    out_hbm.at[pl.ds(c * NP + s * RPT, RPT)])


@functools.partial(
    pl.kernel,
    out_type=jax.ShapeDtypeStruct((NC * NP, H), jnp.float32),
    mesh=_MESH,
    scratch_types=[
        pltpu.VMEM((2, WIN, CHUNK), jnp.int32),
        pltpu.VMEM((2, WIN, CHUNK), jnp.int32),
        pltpu.VMEM((2, CHUNK, H), jnp.float32),
        pltpu.VMEM_SHARED((NP, H), jnp.float32),
        pltpu.SemaphoreType.DMA,
        pltpu.SemaphoreType.DMA,
        pltpu.SemaphoreType.DMA,
        pltpu.SemaphoreType.DMA,
        pltpu.SemaphoreType.DMA,
        pltpu.SemaphoreType.DMA,
    ],
)
def _segsum_sc(g_hbm, src_hbm, dst_hbm, zeros_hbm, out_hbm,
               idx_s, idx_d, rows, acc, g0, g1, t0, t1, is_sem, id_sem):
    c = lax.axis_index("c")
    s = lax.axis_index("s")
    w = c * NS + s
    base = w * NCHUNK
    gsem = [g0, g1]
    ssem = [t0, t1]

    pltpu.sync_copy(src_hbm.at[pl.ds(base, WIN)], idx_s.at[0])
    pltpu.sync_copy(dst_hbm.at[pl.ds(base, WIN)], idx_d.at[0])
    pltpu.sync_copy(zeros_hbm.at[pl.ds(s * RPT, RPT)], acc.at[pl.ds(s * RPT, RPT)])
    plsc.subcore_barrier()

    pltpu.async_copy(g_hbm.at[idx_s.at[0, 0]], rows.at[0], g0)

    def win_body(wi, carry):
        par = lax.rem(wi, 2)
        nxt = 1 - par

        @pl.when(wi >= 1)
        def _():
            pltpu.make_async_copy(
                rows.at[1], acc.at[idx_d.at[nxt, WIN - 1]], ssem[1]).wait()

        @pl.when(wi + 1 < NWIN)
        def _():
            pltpu.async_copy(src_hbm.at[pl.ds(base + (wi + 1) * WIN, WIN)],
                             idx_s.at[nxt], is_sem)
            pltpu.async_copy(dst_hbm.at[pl.ds(base + (wi + 1) * WIN, WIN)],
                             idx_d.at[nxt], id_sem)

        pltpu.async_copy(g_hbm.at[idx_s.at[par, 1]], rows.at[1], gsem[1])
        pltpu.make_async_copy(
            g_hbm.at[idx_s.at[par, 0]], rows.at[0], gsem[0]).wait()
        pltpu.async_copy(rows.at[0], acc.at[idx_d.at[par, 0]], ssem[0],
                         add=True)

        for j in range(1, WIN):
            b = j % 2
            o = 1 - b
            pltpu.make_async_copy(
                rows.at[o], acc.at[idx_d.at[par, j - 1]], ssem[o]).wait()
            if j < WIN - 1:
                pltpu.async_copy(g_hbm.at[idx_s.at[par, j + 1]],
                                 rows.at[o], gsem[o])
            else:
                @pl.when(wi + 1 < NWIN)
                def _():
                    pltpu.async_copy(g_hbm.at[idx_s.at[nxt, 0]],
                                     rows.at[o], gsem[o])
            pltpu.make_async_copy(
                g_hbm.at[idx_s.at[par, j]], rows.at[b], gsem[b]).wait()
            pltpu.async_copy(rows.at[b], acc.at[idx_d.at[par, j]], ssem[b],
                             add=True)
            if j == WIN - 2:
                @pl.when(wi + 1 < NWIN)
                def _():
                    pltpu.make_async_copy(
                        src_hbm.at[pl.ds(base + (wi + 1) * WIN, WIN)],
                        idx_s.at[nxt], is_sem).wait()
                    pltpu.make_async_copy(
                        dst_hbm.at[pl.ds(base + (wi + 1) * WIN, WIN)],
                        idx_d.at[nxt], id_sem).wait()
        return carry

    lax.fori_loop(0, NWIN, win_body, 0)
    pltpu.make_async_copy(
        rows.at[1], acc.at[idx_d.at[lax.rem(NWIN - 1, 2), WIN - 1]],
        ssem[1]).wait()
    plsc.subcore_barrier()
    pltpu.sync_copy(acc.at[pl.ds(s * RPT, RPT)],
                    out_hbm.at[pl.ds(c * NP + s * RPT, RPT)])


RB = 1024
_P = lax.Precision.HIGHEST


def _dinv_of(degp_blk):
    deg = degp_blk[0] + degp_blk[1]
    return jnp.where(deg > 0.0, lax.rsqrt(deg), 0.0)[:, None]


def _fuse1_body(x_blk, degp_blk, wego, bego, w1, hego_out, g1_out):
    dinv = _dinv_of(degp_blk)
    xb = x_blk[...]
    hego_out[...] = jnp.maximum(
        jnp.dot(xb, wego[...], precision=_P) + bego[...], 0.0)
    g1_out[...] = dinv * jnp.dot(xb, w1[...], precision=_P)


def _fuse2_body(s1p_blk, degp_blk, b1, w2, h1_out, g2_out):
    dinv = _dinv_of(degp_blk)
    s1 = s1p_blk[0] + s1p_blk[1]
    h1 = jnp.maximum(dinv * s1 + b1[...], 0.0)
    h1_out[...] = h1
    g2_out[...] = dinv * jnp.dot(h1, w2[...], precision=_P)


def _fuse3_body(s2p_blk, degp_blk, b2, hego_blk, h1_blk, wc, bc, out_blk):
    dinv = _dinv_of(degp_blk)
    h2 = jnp.maximum(dinv * (s2p_blk[0] + s2p_blk[1]) + b2[...], 0.0)
    wcr = wc[...]
    out_blk[...] = (
        jnp.dot(hego_blk[...], wcr[0:H], precision=_P)
        + jnp.dot(h1_blk[...], wcr[H:2 * H], precision=_P)
        + jnp.dot(h2, wcr[2 * H:3 * H], precision=_P)
        + bc[...])


def _row_spec(width):
    return pl.BlockSpec((RB, width), lambda i: (i, 0))


def _part_spec(width):
    return pl.BlockSpec((2, RB, width), lambda i: (0, i, 0))


_DEG_SPEC = pl.BlockSpec((2, RB), lambda i: (0, i))


def _full_spec(shape):
    return pl.BlockSpec(shape, lambda i: tuple(0 for _ in shape))


_GRID = ((N + RB - 1) // RB,)

_fuse1 = pl.pallas_call(
    _fuse1_body,
    grid=_GRID,
    in_specs=[_row_spec(D), _DEG_SPEC, _full_spec((D, H)),
              _full_spec((1, H)), _full_spec((D, H))],
    out_specs=[_row_spec(H), _row_spec(H)],
    out_shape=[jax.ShapeDtypeStruct((N, H), jnp.float32),
               jax.ShapeDtypeStruct((N, H), jnp.float32)],
)

_fuse2 = pl.pallas_call(
    _fuse2_body,
    grid=_GRID,
    in_specs=[_part_spec(H), _DEG_SPEC, _full_spec((1, H)),
              _full_spec((H, H))],
    out_specs=[_row_spec(H), _row_spec(H)],
    out_shape=[jax.ShapeDtypeStruct((N, H), jnp.float32),
               jax.ShapeDtypeStruct((N, H), jnp.float32)],
)

_fuse3 = pl.pallas_call(
    _fuse3_body,
    grid=_GRID,
    in_specs=[_part_spec(H), _DEG_SPEC, _full_spec((1, H)),
              _row_spec(H), _row_spec(H), _full_spec((3 * H, C)),
              _full_spec((1, C))],
    out_specs=_row_spec(C),
    out_shape=jax.ShapeDtypeStruct((N, C), jnp.float32),
)


def kernel(x, edge_index, W_ego, b_ego, W1, b1, W2, b2, Wc, bc):
    src = jnp.pad(edge_index[0].reshape(NW, EW), ((0, 0), (0, EWP - EW)),
                  constant_values=0).reshape(NW * NCHUNK, CHUNK)
    dst = jnp.pad(edge_index[1].reshape(NW, EW), ((0, 0), (0, EWP - EW)),
                  constant_values=NP - 1).reshape(NW * NCHUNK, CHUNK)
    zeros_nh = jnp.zeros((NP, H), jnp.float32)
    zeros_nd = jnp.zeros((NP,), jnp.float32)
    ones_cd = jnp.ones((CHUNK,), jnp.float32)

    degp = _deg_sc(dst, ones_cd, zeros_nd).reshape(NC, NP)
    hego, g1 = _fuse1(x, degp, W_ego, b_ego.reshape(1, H), W1)
    s1p = _segsum_sc(g1, src, dst, zeros_nh).reshape(NC, NP, H)
    h1, g2 = _fuse2(s1p, degp, b1.reshape(1, H), W2)
    s2p = _segsum_sc(g2, src, dst, zeros_nh).reshape(NC, NP, H)
    return _fuse3(s2p, degp, b2.reshape(1, H), hego, h1, Wc,
                  bc.reshape(1, C))


# speedup vs baseline: 2.7801x; 2.7801x over previous
import functools

import jax
import jax.numpy as jnp
from jax import lax
from jax.experimental import pallas as pl
from jax.experimental.pallas import tpu as pltpu
from jax.experimental.pallas import tpu_sc as plsc

N = 10000
E = 320000
D = 128
H = 128
C = 64

NC = 2
NS = 16
NW = NC * NS
EW = E // NW
CHUNK = 128
NCHUNK = 80
EWP = NCHUNK * CHUNK
WIN = 8
NWIN = NCHUNK // WIN
NP = 10240
RPT = NP // NS

_MESH = plsc.VectorSubcoreMesh(
    core_axis_name="c", subcore_axis_name="s", num_cores=NC, num_subcores=NS)


@functools.partial(
    pl.kernel,
    out_type=jax.ShapeDtypeStruct((NC * NP,), jnp.float32),
    mesh=_MESH,
    scratch_types=[
        pltpu.VMEM((NCHUNK, CHUNK), jnp.int32),
        pltpu.VMEM((CHUNK,), jnp.float32),
        pltpu.VMEM_SHARED((NP,), jnp.float32),
    ],
)
def _deg_sc(dst_hbm, ones_hbm, zeros_hbm, out_hbm, idx_d, ones_v, acc):
    c = lax.axis_index("c")
    s = lax.axis_index("s")
    w = c * NS + s
    pltpu.sync_copy(ones_hbm, ones_v)
    pltpu.sync_copy(dst_hbm.at[pl.ds(w * NCHUNK, NCHUNK)], idx_d)
    pltpu.sync_copy(zeros_hbm.at[pl.ds(s * RPT, RPT)], acc.at[pl.ds(s * RPT, RPT)])
    plsc.subcore_barrier()

    def body(i, carry):
        pltpu.sync_copy(ones_v, acc.at[idx_d.at[i]], add=True)
        return carry

    lax.fori_loop(0, NCHUNK, body, 0)
    plsc.subcore_barrier()
    pltpu.sync_copy(acc.at[pl.ds(s * RPT, RPT)],
                    out_hbm.at[pl.ds(c * NP + s * RPT, RPT)])


@functools.partial(
    pl.kernel,
    out_type=jax.ShapeDtypeStruct((NC * NP, H), jnp.float32),
    mesh=_MESH,
    scratch_types=[
        pltpu.VMEM((2, WIN, CHUNK), jnp.int32),
        pltpu.VMEM((2, WIN, CHUNK), jnp.int32),
        pltpu.VMEM((2, CHUNK, H), jnp.float32),
        pltpu.VMEM_SHARED((NP, H), jnp.float32),
        pltpu.SemaphoreType.DMA,
        pltpu.SemaphoreType.DMA,
        pltpu.SemaphoreType.DMA,
        pltpu.SemaphoreType.DMA,
        pltpu.SemaphoreType.DMA,
        pltpu.SemaphoreType.DMA,
    ],
)
def _segsum_sc(g_hbm, src_hbm, dst_hbm, zeros_hbm, out_hbm,
               idx_s, idx_d, rows, acc, g0, g1, t0, t1, is_sem, id_sem):
    c = lax.axis_index("c")
    s = lax.axis_index("s")
    w = c * NS + s
    base = w * NCHUNK
    gsem = [g0, g1]
    ssem = [t0, t1]

    pltpu.sync_copy(src_hbm.at[pl.ds(base, WIN)], idx_s.at[0])
    pltpu.sync_copy(dst_hbm.at[pl.ds(base, WIN)], idx_d.at[0])
    pltpu.sync_copy(zeros_hbm.at[pl.ds(s * RPT, RPT)], acc.at[pl.ds(s * RPT, RPT)])
    plsc.subcore_barrier()

    pltpu.async_copy(g_hbm.at[idx_s.at[0, 0]], rows.at[0], g0)

    def win_body(wi, carry):
        par = lax.rem(wi, 2)
        nxt = 1 - par

        @pl.when(wi >= 1)
        def _():
            pltpu.make_async_copy(
                rows.at[1], acc.at[idx_d.at[nxt, WIN - 1]], ssem[1]).wait()

        @pl.when(wi + 1 < NWIN)
        def _():
            pltpu.async_copy(src_hbm.at[pl.ds(base + (wi + 1) * WIN, WIN)],
                             idx_s.at[nxt], is_sem)
            pltpu.async_copy(dst_hbm.at[pl.ds(base + (wi + 1) * WIN, WIN)],
                             idx_d.at[nxt], id_sem)

        pltpu.async_copy(g_hbm.at[idx_s.at[par, 1]], rows.at[1], gsem[1])
        pltpu.make_async_copy(
            g_hbm.at[idx_s.at[par, 0]], rows.at[0], gsem[0]).wait()
        pltpu.async_copy(rows.at[0], acc.at[idx_d.at[par, 0]], ssem[0],
                         add=True)

        for j in range(1, WIN):
            b = j % 2
            o = 1 - b
            pltpu.make_async_copy(
                rows.at[o], acc.at[idx_d.at[par, j - 1]], ssem[o]).wait()
            if j < WIN - 1:
                pltpu.async_copy(g_hbm.at[idx_s.at[par, j + 1]],
                                 rows.at[o], gsem[o])
            else:
                @pl.when(wi + 1 < NWIN)
                def _():
                    pltpu.async_copy(g_hbm.at[idx_s.at[nxt, 0]],
                                     rows.at[o], gsem[o])
            pltpu.make_async_copy(
                g_hbm.at[idx_s.at[par, j]], rows.at[b], gsem[b]).wait()
            pltpu.async_copy(rows.at[b], acc.at[idx_d.at[par, j]], ssem[b],
                             add=True)
            if j == WIN - 2:
                @pl.when(wi + 1 < NWIN)
                def _():
                    pltpu.make_async_copy(
                        src_hbm.at[pl.ds(base + (wi + 1) * WIN, WIN)],
                        idx_s.at[nxt], is_sem).wait()
                    pltpu.make_async_copy(
                        dst_hbm.at[pl.ds(base + (wi + 1) * WIN, WIN)],
                        idx_d.at[nxt], id_sem).wait()
        return carry

    lax.fori_loop(0, NWIN, win_body, 0)
    pltpu.make_async_copy(
        rows.at[1], acc.at[idx_d.at[lax.rem(NWIN - 1, 2), WIN - 1]],
        ssem[1]).wait()
    plsc.subcore_barrier()
    pltpu.sync_copy(acc.at[pl.ds(s * RPT, RPT)],
                    out_hbm.at[pl.ds(c * NP + s * RPT, RPT)])


RB = 1024
_P = lax.Precision.HIGHEST


def _dinv_of(degp_blk):
    deg = degp_blk[0] + degp_blk[1]
    return jnp.where(deg > 0.0, lax.rsqrt(deg), 0.0)[:, None]


def _fuse1_body(x_blk, degp_blk, wego, bego, w1, hego_out, g1_out):
    dinv = _dinv_of(degp_blk)
    xb = x_blk[...]
    hego_out[...] = jnp.maximum(
        jnp.dot(xb, wego[...], precision=_P) + bego[...], 0.0)
    g1_out[...] = dinv * jnp.dot(xb, w1[...], precision=_P)


def _fuse2_body(s1p_blk, degp_blk, b1, w2, h1_out, g2_out):
    dinv = _dinv_of(degp_blk)
    s1 = s1p_blk[0] + s1p_blk[1]
    h1 = jnp.maximum(dinv * s1 + b1[...], 0.0)
    h1_out[...] = h1
    g2_out[...] = dinv * jnp.dot(h1, w2[...], precision=_P)


def _fuse3_body(s2p_blk, degp_blk, b2, hego_blk, h1_blk, wc, bc, out_blk):
    dinv = _dinv_of(degp_blk)
    h2 = jnp.maximum(dinv * (s2p_blk[0] + s2p_blk[1]) + b2[...], 0.0)
    wcr = wc[...]
    out_blk[...] = (
        jnp.dot(hego_blk[...], wcr[0:H], precision=_P)
        + jnp.dot(h1_blk[...], wcr[H:2 * H], precision=_P)
        + jnp.dot(h2, wcr[2 * H:3 * H], precision=_P)
        + bc[...])


def _row_spec(width):
    return pl.BlockSpec((RB, width), lambda i: (i, 0))


def _part_spec(width):
    return pl.BlockSpec((2, RB, width), lambda i: (0, i, 0))


_DEG_SPEC = pl.BlockSpec((2, RB), lambda i: (0, i))


def _full_spec(shape):
    return pl.BlockSpec(shape, lambda i: tuple(0 for _ in shape))


_GRID = ((N + RB - 1) // RB,)

_fuse1 = pl.pallas_call(
    _fuse1_body,
    grid=_GRID,
    in_specs=[_row_spec(D), _DEG_SPEC, _full_spec((D, H)),
              _full_spec((1, H)), _full_spec((D, H))],
    out_specs=[_row_spec(H), _row_spec(H)],
    out_shape=[jax.ShapeDtypeStruct((N, H), jnp.float32),
               jax.ShapeDtypeStruct((N, H), jnp.float32)],
)

_fuse2 = pl.pallas_call(
    _fuse2_body,
    grid=_GRID,
    in_specs=[_part_spec(H), _DEG_SPEC, _full_spec((1, H)),
              _full_spec((H, H))],
    out_specs=[_row_spec(H), _row_spec(H)],
    out_shape=[jax.ShapeDtypeStruct((N, H), jnp.float32),
               jax.ShapeDtypeStruct((N, H), jnp.float32)],
)

_fuse3 = pl.pallas_call(
    _fuse3_body,
    grid=_GRID,
    in_specs=[_part_spec(H), _DEG_SPEC, _full_spec((1, H)),
              _row_spec(H), _row_spec(H), _full_spec((3 * H, C)),
              _full_spec((1, C))],
    out_specs=_row_spec(C),
    out_shape=jax.ShapeDtypeStruct((N, C), jnp.float32),
)


def kernel(x, edge_index, W_ego, b_ego, W1, b1, W2, b2, Wc, bc):
    npad = EWP - EW
    lane = jnp.arange(npad, dtype=jnp.int32)[None, :]
    tile = jnp.arange(NW, dtype=jnp.int32)[:, None]
    pad_src = (tile * npad + lane) % N
    pad_dst = N + (tile * 7 + lane) % (NP - N)
    src = jnp.concatenate([edge_index[0].reshape(NW, EW), pad_src],
                          axis=1).reshape(NW * NCHUNK, CHUNK)
    dst = jnp.concatenate([edge_index[1].reshape(NW, EW), pad_dst],
                          axis=1).reshape(NW * NCHUNK, CHUNK)
    zeros_nh = jnp.zeros((NP, H), jnp.float32)
    zeros_nd = jnp.zeros((NP,), jnp.float32)
    ones_cd = jnp.ones((CHUNK,), jnp.float32)

    degp = _deg_sc(dst, ones_cd, zeros_nd).reshape(NC, NP)
    hego, g1 = _fuse1(x, degp, W_ego, b_ego.reshape(1, H), W1)
    s1p = _segsum_sc(g1, src, dst, zeros_nh).reshape(NC, NP, H)
    h1, g2 = _fuse2(s1p, degp, b1.reshape(1, H), W2)
    s2p = _segsum_sc(g2, src, dst, zeros_nh).reshape(NC, NP, H)
    return _fuse3(s2p, degp, b2.reshape(1, H), hego, h1, Wc,
                  bc.reshape(1, C))
